# baseline (speedup 1.0000x reference)
import functools

import jax
import jax.numpy as jnp
from jax import lax
from jax.experimental import pallas as pl
from jax.experimental.pallas import tpu as pltpu


def _round_up(x, m):
    return (x + m - 1) // m * m


def _emb_ln_kernel(ids_ref,
                   word_hbm,
                   pos_ref,
                   gamma_ref,
                   beta_ref,
                   out_ref,
                   wbuf,
                   sem,
                   *, tq, eps):
    c = pl.program_id(0)
    t = pl.program_id(1)
    n_t = pl.num_programs(1)
    slot = lax.rem(t, 2)

    def start_gather(tile, slot_):
        base = tile * tq
        for r in range(tq):
            tok = ids_ref[base + r]
            pltpu.make_async_copy(
                word_hbm.at[pl.ds(tok, 1), :],
                wbuf.at[slot_, pl.ds(r, 1), :],
                sem.at[slot_],
            ).start(priority=r % 2)

    @pl.when(t == 0)
    def _():
        start_gather(c * n_t, slot)

    @pl.when(t + 1 < n_t)
    def _():
        start_gather(c * n_t + t + 1, 1 - slot)

    pltpu.make_async_copy(
        word_hbm.at[pl.ds(0, tq), :],
        wbuf.at[slot],
        sem.at[slot],
    ).wait()

    x = wbuf[slot] + pos_ref[...]
    mean = jnp.mean(x, axis=-1, keepdims=True)
    xc = x - mean
    var = jnp.mean(xc * xc, axis=-1, keepdims=True)
    scale = lax.rsqrt(var + eps) * gamma_ref[...]
    out_ref[...] = xc * scale + beta_ref[...]


def _embeddings(input_ids, word_emb, pos_emb, ln_gamma, ln_beta,
                eps=1e-12, rows_per_step=2):
    bs, seq = input_ids.shape
    vocab, d = word_emb.shape

    seq_p = _round_up(seq, 8)
    n_c = 2 if bs % 2 == 0 else 1
    n_b = bs // n_c
    g = rows_per_step if n_b % rows_per_step == 0 else 1
    tq = g * seq_p
    n_t = n_b // g

    ids = input_ids.astype(jnp.int32)
    if seq_p != seq:
        ids = jnp.pad(ids, ((0, 0), (0, seq_p - seq)))
    ids_flat = ids.reshape(bs * seq_p)

    pos_tab = pos_emb
    if pos_tab.shape[0] < seq_p:
        pos_tab = jnp.pad(pos_tab, ((0, seq_p - pos_tab.shape[0]), (0, 0)))
    elif pos_tab.shape[0] > seq_p:
        pos_tab = pos_tab[:seq_p]
    if g > 1:
        pos_tab = jnp.tile(pos_tab, (g, 1))

    gamma2 = ln_gamma.reshape(1, d)
    beta2 = ln_beta.reshape(1, d)

    grid_spec = pltpu.PrefetchScalarGridSpec(
        num_scalar_prefetch=1,
        grid=(n_c, n_t),
        in_specs=[
            pl.BlockSpec(memory_space=pl.ANY),
            pl.BlockSpec((tq, d), lambda c, t, ids_smem: (0, 0)),
            pl.BlockSpec((1, d), lambda c, t, ids_smem: (0, 0)),
            pl.BlockSpec((1, d), lambda c, t, ids_smem: (0, 0)),
        ],
        out_specs=pl.BlockSpec(
            (tq, d), lambda c, t, ids_smem: (c * n_t + t, 0)),
        scratch_shapes=[
            pltpu.VMEM((2, tq, d), jnp.float32),
            pltpu.SemaphoreType.DMA((2,)),
        ],
    )

    out_flat = pl.pallas_call(
        functools.partial(_emb_ln_kernel, tq=tq, eps=eps),
        grid_spec=grid_spec,
        out_shape=jax.ShapeDtypeStruct((bs * seq_p, d), jnp.float32),
        compiler_params=pltpu.CompilerParams(
            dimension_semantics=("parallel", "arbitrary"),
            disable_bounds_checks=True,
        ),
    )(ids_flat, word_emb, pos_tab, gamma2, beta2)

    out = out_flat.reshape(bs, seq_p, d)
    if seq_p != seq:
        out = out[:, :seq, :]
    return out


def kernel(input_ids, word_emb, pos_emb, ln_gamma, ln_beta):
    return _embeddings(input_ids, word_emb, pos_emb, ln_gamma, ln_beta)


# speedup vs baseline: 1.0131x; 1.0131x over previous
import functools

import jax
import jax.numpy as jnp
from jax import lax
from jax.experimental import pallas as pl
from jax.experimental.pallas import tpu as pltpu


def _round_up(x, m):
    return (x + m - 1) // m * m


def _emb_ln_kernel(ids_ref,
                   word_hbm,
                   pos_ref,
                   gamma_ref,
                   beta_ref,
                   out_ref,
                   wbuf,
                   sem,
                   *, tq, eps):
    c = pl.program_id(0)
    t = pl.program_id(1)
    n_t = pl.num_programs(1)
    slot = lax.rem(t, 2)

    def start_gather(tile, slot_):
        base = tile * tq
        for r in range(tq):
            tok = ids_ref[base + r]
            pltpu.make_async_copy(
                word_hbm.at[pl.ds(tok, 1), :],
                wbuf.at[slot_, pl.ds(r, 1), :],
                sem.at[slot_],
            ).start(priority=r % 2)

    @pl.when(t == 0)
    def _():
        start_gather(c * n_t, slot)

    @pl.when(t + 1 < n_t)
    def _():
        start_gather(c * n_t + t + 1, 1 - slot)

    pltpu.make_async_copy(
        word_hbm.at[pl.ds(0, tq), :],
        wbuf.at[slot],
        sem.at[slot],
    ).wait()

    x = wbuf[slot] + pos_ref[...]
    mean = jnp.mean(x, axis=-1, keepdims=True)
    xc = x - mean
    var = jnp.mean(xc * xc, axis=-1, keepdims=True)
    scale = lax.rsqrt(var + eps) * gamma_ref[...]
    out_ref[...] = xc * scale + beta_ref[...]


def _embeddings(input_ids, word_emb, pos_emb, ln_gamma, ln_beta,
                eps=1e-12, rows_per_step=1):
    bs, seq = input_ids.shape
    vocab, d = word_emb.shape

    seq_p = _round_up(seq, 8)
    n_c = 2 if bs % 2 == 0 else 1
    n_b = bs // n_c
    g = rows_per_step if n_b % rows_per_step == 0 else 1
    tq = g * seq_p
    n_t = n_b // g

    ids = input_ids.astype(jnp.int32)
    if seq_p != seq:
        ids = jnp.pad(ids, ((0, 0), (0, seq_p - seq)))
    ids_flat = ids.reshape(bs * seq_p)

    pos_tab = pos_emb
    if pos_tab.shape[0] < seq_p:
        pos_tab = jnp.pad(pos_tab, ((0, seq_p - pos_tab.shape[0]), (0, 0)))
    elif pos_tab.shape[0] > seq_p:
        pos_tab = pos_tab[:seq_p]
    if g > 1:
        pos_tab = jnp.tile(pos_tab, (g, 1))

    gamma2 = ln_gamma.reshape(1, d)
    beta2 = ln_beta.reshape(1, d)

    grid_spec = pltpu.PrefetchScalarGridSpec(
        num_scalar_prefetch=1,
        grid=(n_c, n_t),
        in_specs=[
            pl.BlockSpec(memory_space=pl.ANY),
            pl.BlockSpec((tq, d), lambda c, t, ids_smem: (0, 0)),
            pl.BlockSpec((1, d), lambda c, t, ids_smem: (0, 0)),
            pl.BlockSpec((1, d), lambda c, t, ids_smem: (0, 0)),
        ],
        out_specs=pl.BlockSpec(
            (tq, d), lambda c, t, ids_smem: (c * n_t + t, 0)),
        scratch_shapes=[
            pltpu.VMEM((2, tq, d), jnp.float32),
            pltpu.SemaphoreType.DMA((2,)),
        ],
    )

    out_flat = pl.pallas_call(
        functools.partial(_emb_ln_kernel, tq=tq, eps=eps),
        grid_spec=grid_spec,
        out_shape=jax.ShapeDtypeStruct((bs * seq_p, d), jnp.float32),
        compiler_params=pltpu.CompilerParams(
            dimension_semantics=("parallel", "arbitrary"),
            disable_bounds_checks=True,
        ),
    )(ids_flat, word_emb, pos_tab, gamma2, beta2)

    out = out_flat.reshape(bs, seq_p, d)
    if seq_p != seq:
        out = out[:, :seq, :]
    return out


def kernel(input_ids, word_emb, pos_emb, ln_gamma, ln_beta):
    return _embeddings(input_ids, word_emb, pos_emb, ln_gamma, ln_beta)
